# baseline (speedup 1.0000x reference)
import functools

import jax
import jax.numpy as jnp
from jax.experimental import pallas as pl

STATE_DIM = 256
POCKET_DIM = 64
MAX_STATES = 64
MIN_STATES = 16
B = 1024
N_POCKET = 512
NP = N_POCKET * POCKET_DIM

B_TILE = 128
N_CHUNKS = 4
B_CHUNK = B // N_CHUNKS


def _prep_kernel(mol, base, sW1, sb1, sW2, sb2, dW1, db1, dW2, db2,
                 probs_out, p16_out, s3_out):
    h = jax.nn.silu(
        jnp.dot(mol[...], sW1[...], preferred_element_type=jnp.float32)
        + sb1[...])
    logits = (jnp.dot(h, sW2[...], preferred_element_type=jnp.float32)
              + sb2[...])
    col = jax.lax.broadcasted_iota(jnp.int32, logits.shape, 1)
    masked = jnp.where(col < MIN_STATES, logits, -jnp.inf)
    m = jnp.max(masked, axis=1, keepdims=True)
    e = jnp.exp(masked - m)
    p = e / jnp.sum(e, axis=1, keepdims=True)
    probs_out[...] = p
    p16_out[...] = p[:, :MIN_STATES]

    base_v = base[...]
    for k in range(MIN_STATES):
        h1 = jax.nn.silu(
            jnp.dot(base_v, dW1[k], preferred_element_type=jnp.float32)
            + db1[k:k + 1, :])
        d = (jnp.dot(h1, dW2[k], preferred_element_type=jnp.float32)
             + db2[k:k + 1, :])
        s3_out[k, :, :] = base_v + 0.1 * d


def _mix_kernel(p16, s2, out):
    out[...] = jnp.dot(p16[...], s2[...], preferred_element_type=jnp.float32)


@functools.partial(jax.jit, static_argnames=())
def kernel(mol_embedding, base_pocket, sel_W1, sel_b1, sel_W2, sel_b2,
           def_W1, def_b1, def_W2, def_b2):
    probs, p16, s3 = pl.pallas_call(
        _prep_kernel,
        out_shape=[
            jax.ShapeDtypeStruct((B, MAX_STATES), jnp.float32),
            jax.ShapeDtypeStruct((B, MIN_STATES), jnp.float32),
            jax.ShapeDtypeStruct((MIN_STATES, N_POCKET, POCKET_DIM),
                                 jnp.float32),
        ],
    )(mol_embedding, base_pocket, sel_W1, sel_b1.reshape(1, -1), sel_W2,
      sel_b2.reshape(1, -1), def_W1, def_b1, def_W2, def_b2)

    s2 = s3.reshape(MIN_STATES, NP)
    mix = pl.pallas_call(
        _mix_kernel,
        grid=(B_CHUNK // B_TILE,),
        in_specs=[
            pl.BlockSpec((B_TILE, MIN_STATES), lambda i: (i, 0)),
            pl.BlockSpec((MIN_STATES, NP), lambda i: (0, 0)),
        ],
        out_specs=pl.BlockSpec((B_TILE, NP), lambda i: (i, 0)),
        out_shape=jax.ShapeDtypeStruct((B_CHUNK, NP), jnp.float32),
    )
    parts = []
    for c in range(N_CHUNKS):
        wp2_c = mix(jax.lax.slice(p16, (c * B_CHUNK, 0),
                                  ((c + 1) * B_CHUNK, MIN_STATES)), s2)
        parts.append(wp2_c.reshape(B_CHUNK, N_POCKET, POCKET_DIM))
    weighted_pocket = jnp.concatenate(parts, axis=0)
    active_indices = jnp.arange(MIN_STATES, dtype=jnp.int32)
    return weighted_pocket, probs, active_indices


# speedup vs baseline: 6.0554x; 6.0554x over previous
import functools

import jax
import jax.numpy as jnp
from jax.experimental import pallas as pl

STATE_DIM = 256
POCKET_DIM = 64
MAX_STATES = 64
MIN_STATES = 16
B = 1024
N_POCKET = 512
NP = N_POCKET * POCKET_DIM

B_TILE = 128
NB = B // B_TILE


def _prep_kernel(mol, base, sW1, sb1, sW2, sb2, dW1, db1, dW2, db2,
                 probs_out, p16_out, s3t_out):
    h = jax.nn.silu(
        jnp.dot(mol[...], sW1[...], preferred_element_type=jnp.float32)
        + sb1[...])
    logits = (jnp.dot(h, sW2[...], preferred_element_type=jnp.float32)
              + sb2[...])
    col = jax.lax.broadcasted_iota(jnp.int32, logits.shape, 1)
    masked = jnp.where(col < MIN_STATES, logits, -jnp.inf)
    m = jnp.max(masked, axis=1, keepdims=True)
    e = jnp.exp(masked - m)
    p = e / jnp.sum(e, axis=1, keepdims=True)
    probs_out[...] = p
    p16_out[...] = p[:, :MIN_STATES]

    base_v = base[...]
    for k in range(MIN_STATES):
        h1 = jax.nn.silu(
            jnp.dot(base_v, dW1[k], preferred_element_type=jnp.float32)
            + db1[k:k + 1, :])
        d = (jnp.dot(h1, dW2[k], preferred_element_type=jnp.float32)
             + db2[k:k + 1, :])
        s3t_out[k, :, :] = jnp.transpose(base_v + 0.1 * d)


def _mix_kernel(p16, s2t, out):
    rhs = s2t[...].reshape(MIN_STATES, NP)
    res = jnp.dot(p16[...], rhs, preferred_element_type=jnp.float32)
    out[...] = res.reshape(out.shape)


@functools.partial(jax.jit, static_argnames=())
def kernel(mol_embedding, base_pocket, sel_W1, sel_b1, sel_W2, sel_b2,
           def_W1, def_b1, def_W2, def_b2):
    probs, p16, s3t = pl.pallas_call(
        _prep_kernel,
        out_shape=[
            jax.ShapeDtypeStruct((B, MAX_STATES), jnp.float32),
            jax.ShapeDtypeStruct((B, MIN_STATES), jnp.float32),
            jax.ShapeDtypeStruct((MIN_STATES, POCKET_DIM, N_POCKET),
                                 jnp.float32),
        ],
    )(mol_embedding, base_pocket, sel_W1, sel_b1.reshape(1, -1), sel_W2,
      sel_b2.reshape(1, -1), def_W1, def_b1, def_W2, def_b2)

    wp_t = pl.pallas_call(
        _mix_kernel,
        grid=(NB,),
        in_specs=[
            pl.BlockSpec((B_TILE, MIN_STATES), lambda i: (i, 0)),
            pl.BlockSpec((MIN_STATES, POCKET_DIM, N_POCKET),
                         lambda i: (0, 0, 0)),
        ],
        out_specs=pl.BlockSpec((B_TILE, POCKET_DIM, N_POCKET),
                               lambda i: (i, 0, 0)),
        out_shape=jax.ShapeDtypeStruct((B, POCKET_DIM, N_POCKET),
                                       jnp.float32),
    )(p16, s3t)

    weighted_pocket = wp_t.transpose(0, 2, 1)
    active_indices = jnp.arange(MIN_STATES, dtype=jnp.int32)
    return weighted_pocket, probs, active_indices


# speedup vs baseline: 6.4365x; 1.0629x over previous
import functools

import jax
import jax.numpy as jnp
from jax.experimental import pallas as pl
from jax.experimental.pallas import tpu as pltpu

STATE_DIM = 256
POCKET_DIM = 64
MAX_STATES = 64
MIN_STATES = 16
B = 1024
N_POCKET = 512
NP = N_POCKET * POCKET_DIM

B_TILE = 128
NB = B // B_TILE


def _fused_kernel(mol, base, sW1, sb1, sW2, sb2, dW1, db1, dW2, db2,
                  probs_t_out, wp_t_out, s2t):
    i = pl.program_id(0)

    @pl.when(i == 0)
    def _():
        base_v = base[...]
        for k in range(MIN_STATES):
            h1 = jax.nn.silu(
                jnp.dot(base_v, dW1[k], preferred_element_type=jnp.float32)
                + db1[k:k + 1, :])
            d = (jnp.dot(h1, dW2[k], preferred_element_type=jnp.float32)
                 + db2[k:k + 1, :])
            s2t[k, :, :] = jnp.transpose(base_v + 0.1 * d)

    h = jax.nn.silu(
        jnp.dot(mol[...], sW1[...], preferred_element_type=jnp.float32)
        + sb1[...])
    logits = (jnp.dot(h, sW2[...], preferred_element_type=jnp.float32)
              + sb2[...])
    col = jax.lax.broadcasted_iota(jnp.int32, logits.shape, 1)
    masked = jnp.where(col < MIN_STATES, logits, -jnp.inf)
    m = jnp.max(masked, axis=1, keepdims=True)
    e = jnp.exp(masked - m)
    p = e / jnp.sum(e, axis=1, keepdims=True)
    probs_t_out[...] = jnp.transpose(p)

    rhs = s2t[...].reshape(MIN_STATES, NP)
    res = jnp.dot(p[:, :MIN_STATES], rhs,
                  preferred_element_type=jnp.float32)
    wp_t_out[...] = res.reshape(wp_t_out.shape)


@functools.partial(jax.jit, static_argnames=())
def kernel(mol_embedding, base_pocket, sel_W1, sel_b1, sel_W2, sel_b2,
           def_W1, def_b1, def_W2, def_b2):
    probs_t, wp_t = pl.pallas_call(
        _fused_kernel,
        grid=(NB,),
        in_specs=[
            pl.BlockSpec((B_TILE, STATE_DIM), lambda i: (i, 0)),
            pl.BlockSpec((N_POCKET, POCKET_DIM), lambda i: (0, 0)),
            pl.BlockSpec((STATE_DIM, STATE_DIM), lambda i: (0, 0)),
            pl.BlockSpec((1, STATE_DIM), lambda i: (0, 0)),
            pl.BlockSpec((STATE_DIM, MAX_STATES), lambda i: (0, 0)),
            pl.BlockSpec((1, MAX_STATES), lambda i: (0, 0)),
            pl.BlockSpec((MIN_STATES, POCKET_DIM, STATE_DIM),
                         lambda i: (0, 0, 0)),
            pl.BlockSpec((MIN_STATES, STATE_DIM), lambda i: (0, 0)),
            pl.BlockSpec((MIN_STATES, STATE_DIM, POCKET_DIM),
                         lambda i: (0, 0, 0)),
            pl.BlockSpec((MIN_STATES, POCKET_DIM), lambda i: (0, 0)),
        ],
        out_specs=[
            pl.BlockSpec((MAX_STATES, B_TILE), lambda i: (0, i)),
            pl.BlockSpec((B_TILE, POCKET_DIM, N_POCKET),
                         lambda i: (i, 0, 0)),
        ],
        out_shape=[
            jax.ShapeDtypeStruct((MAX_STATES, B), jnp.float32),
            jax.ShapeDtypeStruct((B, POCKET_DIM, N_POCKET), jnp.float32),
        ],
        scratch_shapes=[
            pltpu.VMEM((MIN_STATES, POCKET_DIM, N_POCKET), jnp.float32),
        ],
        compiler_params=pltpu.CompilerParams(
            dimension_semantics=("arbitrary",)),
    )(mol_embedding, base_pocket, sel_W1, sel_b1.reshape(1, -1), sel_W2,
      sel_b2.reshape(1, -1), def_W1, def_b1, def_W2, def_b2)

    weighted_pocket = wp_t.transpose(0, 2, 1)
    probs = probs_t.T
    active_indices = jnp.arange(MIN_STATES, dtype=jnp.int32)
    return weighted_pocket, probs, active_indices


# speedup vs baseline: 6.5121x; 1.0118x over previous
import functools

import jax
import jax.numpy as jnp
from jax.experimental import pallas as pl
from jax.experimental.pallas import tpu as pltpu

STATE_DIM = 256
POCKET_DIM = 64
MAX_STATES = 64
MIN_STATES = 16
B = 1024
N_POCKET = 512
NP = N_POCKET * POCKET_DIM

B_TILE = 128
NB = B // B_TILE


def _fused_kernel(mol, base_t, sW1, sb1, sW2t, sb2, dW1, db1_t, dW2t,
                  db2_t, probs_t_out, wp_t_out, s2t):
    i = pl.program_id(0)

    @pl.when(i == 0)
    def _():
        bt = base_t[...]
        for k in range(MIN_STATES):
            h1_t = jax.nn.silu(
                jax.lax.dot_general(
                    dW1[k], bt, (((0,), (0,)), ((), ())),
                    preferred_element_type=jnp.float32)
                + db1_t[:, k:k + 1])
            d_t = (jax.lax.dot_general(
                dW2t[k], h1_t, (((1,), (0,)), ((), ())),
                preferred_element_type=jnp.float32)
                + db2_t[:, k:k + 1])
            s2t[k, :, :] = bt + 0.1 * d_t

    h = jax.nn.silu(
        jnp.dot(mol[...], sW1[...], preferred_element_type=jnp.float32)
        + sb1[...])
    logits = (jax.lax.dot_general(
        h, sW2t[...], (((1,), (1,)), ((), ())),
        preferred_element_type=jnp.float32) + sb2[...])
    col = jax.lax.broadcasted_iota(jnp.int32, logits.shape, 1)
    masked = jnp.where(col < MIN_STATES, logits, -jnp.inf)
    m = jnp.max(masked, axis=1, keepdims=True)
    e = jnp.exp(masked - m)
    p = e / jnp.sum(e, axis=1, keepdims=True)
    probs_t_out[...] = jnp.transpose(p)

    rhs = s2t[...].reshape(MIN_STATES, NP)
    res = jnp.dot(p[:, :MIN_STATES], rhs,
                  preferred_element_type=jnp.float32)
    wp_t_out[...] = res.reshape(wp_t_out.shape)


@functools.partial(jax.jit, static_argnames=())
def kernel(mol_embedding, base_pocket, sel_W1, sel_b1, sel_W2, sel_b2,
           def_W1, def_b1, def_W2, def_b2):
    probs_t, wp_t = pl.pallas_call(
        _fused_kernel,
        grid=(NB,),
        in_specs=[
            pl.BlockSpec((B_TILE, STATE_DIM), lambda i: (i, 0)),
            pl.BlockSpec((POCKET_DIM, N_POCKET), lambda i: (0, 0)),
            pl.BlockSpec((STATE_DIM, STATE_DIM), lambda i: (0, 0)),
            pl.BlockSpec((1, STATE_DIM), lambda i: (0, 0)),
            pl.BlockSpec((MAX_STATES, STATE_DIM), lambda i: (0, 0)),
            pl.BlockSpec((1, MAX_STATES), lambda i: (0, 0)),
            pl.BlockSpec((MIN_STATES, POCKET_DIM, STATE_DIM),
                         lambda i: (0, 0, 0)),
            pl.BlockSpec((STATE_DIM, MIN_STATES), lambda i: (0, 0)),
            pl.BlockSpec((MIN_STATES, POCKET_DIM, STATE_DIM),
                         lambda i: (0, 0, 0)),
            pl.BlockSpec((POCKET_DIM, MIN_STATES), lambda i: (0, 0)),
        ],
        out_specs=[
            pl.BlockSpec((MAX_STATES, B_TILE), lambda i: (0, i)),
            pl.BlockSpec((B_TILE, POCKET_DIM, N_POCKET),
                         lambda i: (i, 0, 0)),
        ],
        out_shape=[
            jax.ShapeDtypeStruct((MAX_STATES, B), jnp.float32),
            jax.ShapeDtypeStruct((B, POCKET_DIM, N_POCKET), jnp.float32),
        ],
        scratch_shapes=[
            pltpu.VMEM((MIN_STATES, POCKET_DIM, N_POCKET), jnp.float32),
        ],
        compiler_params=pltpu.CompilerParams(
            dimension_semantics=("arbitrary",)),
    )(mol_embedding, base_pocket.T, sel_W1, sel_b1.reshape(1, -1),
      sel_W2.T, sel_b2.reshape(1, -1), def_W1, def_b1.T,
      def_W2.transpose(0, 2, 1), def_b2.T)

    weighted_pocket = wp_t.transpose(0, 2, 1)
    probs = probs_t.T
    active_indices = jnp.arange(MIN_STATES, dtype=jnp.int32)
    return weighted_pocket, probs, active_indices
